# baseline (speedup 1.0000x reference)
import functools
import math

import jax
import jax.numpy as jnp
from jax import lax
from jax.experimental import pallas as pl
from jax.experimental.pallas import tpu as pltpu
from jax.experimental.pallas import tpu_sc as plsc

VOCAB = 1000000
DIM = 64
LANES = 16
PW = 128
SCALE = math.sqrt(DIM)

_info = plsc.get_sparse_core_info()
_NC = _info.num_cores
_NW = _NC * _info.num_subcores

_mesh = plsc.VectorSubcoreMesh(core_axis_name="c", subcore_axis_name="s")
_params = pltpu.CompilerParams(
    use_tc_tiling_on_sc=True, needs_layout_passes=False
)

_PBK = 8192


def _pad_body(x_ref, o_ref):
    y = jnp.transpose(x_ref[...]) * jnp.float32(SCALE)
    o_ref[...] = jnp.concatenate(
        [y, jnp.zeros((_PBK, PW - DIM), jnp.float32)], axis=1
    )


_pad_tc = pl.pallas_call(
    _pad_body,
    grid=(pl.cdiv(VOCAB, _PBK),),
    in_specs=[pl.BlockSpec((DIM, _PBK), lambda i: (0, i))],
    out_specs=pl.BlockSpec((_PBK, PW), lambda i: (i, 0)),
    out_shape=jax.ShapeDtypeStruct((VOCAB, PW), jnp.float32),
)


@functools.lru_cache(maxsize=None)
def _make_lookup(BA, T):
    assert BA // 128 == _NW and BA % 128 == 0 and T % 4 == 0

    @functools.partial(
        pl.kernel,
        mesh=_mesh,
        out_type=jax.ShapeDtypeStruct((T, DIM, BA), jnp.float32),
        scratch_types=[
            pltpu.VMEM((T, 128), jnp.int32),
            pltpu.VMEM((128,), jnp.int32),
            pltpu.VMEM((128,), jnp.int32),
            pltpu.VMEM((128,), jnp.int32),
            pltpu.VMEM((128,), jnp.int32),
            pltpu.VMEM((128, PW), jnp.float32),
            pltpu.VMEM((128, PW), jnp.float32),
            pltpu.VMEM((128, PW), jnp.float32),
            pltpu.VMEM((128, PW), jnp.float32),
            pltpu.VMEM((DIM, 128), jnp.float32),
            pltpu.VMEM((DIM, 128), jnp.float32),
            pltpu.SemaphoreType.DMA,
            pltpu.SemaphoreType.DMA,
            pltpu.SemaphoreType.DMA,
            pltpu.SemaphoreType.DMA,
            pltpu.SemaphoreType.DMA,
            pltpu.SemaphoreType.DMA,
        ],
        compiler_params=_params,
    )
    def _lookup(
        idxt_hbm, p_hbm, out_hbm,
        ibig, gb0, gb1, gb2, gb3, rows0, rows1, rows2, rows3, sb0, sb1,
        gs0, gs1, gs2, gs3, ws0, ws1,
    ):
        wid = lax.axis_index("s") * _NC + lax.axis_index("c")
        b0 = pl.multiple_of(wid * 128, 128)
        iota = lax.iota(jnp.int32, LANES)
        colgs = [iota + (g * LANES) for g in range(8)]

        pltpu.sync_copy(idxt_hbm.at[:, pl.ds(b0, 128)], ibig)

        def fetch(t, gb, rows, gs):
            del gb
            pltpu.async_copy(p_hbm.at[ibig.at[t]], rows, gs)

        def gwait(rows, gs):
            pltpu.make_async_copy(p_hbm.at[pl.ds(0, 128)], rows, gs).wait()

        def put(t, sb, ws):
            pltpu.async_copy(sb, out_hbm.at[t, :, pl.ds(b0, 128)], ws)

        def wwait(sb, ws):
            pltpu.make_async_copy(
                sb, out_hbm.at[0, :, pl.ds(b0, 128)], ws
            ).wait()

        def transpose(t, rows, sb):
            del t

            def d_body(k, c):
                for u in range(4):
                    d = k * 4 + u
                    rowv = jnp.bitwise_and(iota + d, DIM - 1)
                    for g in range(8):
                        val = plsc.load_gather(rows, [colgs[g], rowv])
                        plsc.store_scatter(sb, [rowv, colgs[g]], val)
                return c

            lax.fori_loop(0, DIM // 4, d_body, 0)

        rbufs = [rows0, rows1, rows2, rows3]
        gbufs = [gb0, gb1, gb2, gb3]
        gsems = [gs0, gs1, gs2, gs3]
        sbufs = [sb0, sb1]
        wsems = [ws0, ws1]

        for j in range(3):
            fetch(j, gbufs[j], rbufs[j], gsems[j])

        def body(k, carry):
            t = k * 4
            for j in range(4):
                nt = t + j + 3

                @pl.when(nt < T)
                def _(nt=nt, j=j):
                    fetch(
                        nt,
                        gbufs[(j + 3) % 4],
                        rbufs[(j + 3) % 4],
                        gsems[(j + 3) % 4],
                    )

                if j < 2:

                    @pl.when(k > 0)
                    def _(j=j):
                        wwait(sbufs[j], wsems[j])

                else:
                    wwait(sbufs[j % 2], wsems[j % 2])
                gwait(rbufs[j], gsems[j])
                transpose(t + j, rbufs[j], sbufs[j % 2])
                put(t + j, sbufs[j % 2], wsems[j % 2])
            return carry

        lax.fori_loop(0, T // 4, body, 0)
        wwait(sb0, ws0)
        wwait(sb1, ws1)

    return _lookup


def kernel(inp, table):
    ba, t = inp.shape
    idxt = inp.T.astype(jnp.int32)
    tpad = _pad_tc(table.T)
    out3 = _make_lookup(ba, t)(idxt, tpad)
    return jnp.transpose(out3, (2, 0, 1))


# speedup vs baseline: 1.0020x; 1.0020x over previous
import functools
import math

import jax
import jax.numpy as jnp
from jax import lax
from jax.experimental import pallas as pl
from jax.experimental.pallas import tpu as pltpu
from jax.experimental.pallas import tpu_sc as plsc

VOCAB = 1000000
DIM = 64
LANES = 16
PW = 128
SCALE = math.sqrt(DIM)

_info = plsc.get_sparse_core_info()
_NC = _info.num_cores
_NW = _NC * _info.num_subcores

_mesh = plsc.VectorSubcoreMesh(core_axis_name="c", subcore_axis_name="s")
_params = pltpu.CompilerParams(
    use_tc_tiling_on_sc=True, needs_layout_passes=False
)

_PBK = 8192


def _pad_body(x_ref, o_ref):
    y = jnp.transpose(x_ref[...]) * jnp.float32(SCALE)
    o_ref[...] = jnp.concatenate(
        [y, jnp.zeros((_PBK, PW - DIM), jnp.float32)], axis=1
    )


_pad_tc = pl.pallas_call(
    _pad_body,
    grid=(pl.cdiv(VOCAB, _PBK),),
    in_specs=[pl.BlockSpec((DIM, _PBK), lambda i: (0, i))],
    out_specs=pl.BlockSpec((_PBK, PW), lambda i: (i, 0)),
    out_shape=jax.ShapeDtypeStruct((VOCAB, PW), jnp.float32),
)


@functools.lru_cache(maxsize=None)
def _make_lookup(BA, T):
    assert BA // 128 == _NW and BA % 128 == 0 and T % 4 == 0

    @functools.partial(
        pl.kernel,
        mesh=_mesh,
        out_type=jax.ShapeDtypeStruct((T, DIM, BA), jnp.float32),
        scratch_types=[
            pltpu.VMEM((T, 128), jnp.int32),
            pltpu.VMEM((128, PW), jnp.float32),
            pltpu.VMEM((128, PW), jnp.float32),
            pltpu.VMEM((128, PW), jnp.float32),
            pltpu.VMEM((128, PW), jnp.float32),
            pltpu.VMEM((DIM, 128), jnp.float32),
            pltpu.VMEM((DIM, 128), jnp.float32),
            pltpu.SemaphoreType.DMA,
            pltpu.SemaphoreType.DMA,
            pltpu.SemaphoreType.DMA,
            pltpu.SemaphoreType.DMA,
            pltpu.SemaphoreType.DMA,
            pltpu.SemaphoreType.DMA,
        ],
        compiler_params=_params,
    )
    def _lookup(
        idxt_hbm, p_hbm, out_hbm,
        ibig, rows0, rows1, rows2, rows3, sb0, sb1,
        gs0, gs1, gs2, gs3, ws0, ws1,
    ):
        wid = lax.axis_index("s") * _NC + lax.axis_index("c")
        b0 = pl.multiple_of(wid * 128, 128)
        iota = lax.iota(jnp.int32, LANES)
        colgs = [iota + (g * LANES) for g in range(8)]

        pltpu.sync_copy(idxt_hbm.at[:, pl.ds(b0, 128)], ibig)

        def fetch(t, rows, gs):
            pltpu.async_copy(p_hbm.at[ibig.at[t]], rows, gs)

        def gwait(rows, gs):
            pltpu.make_async_copy(p_hbm.at[pl.ds(0, 128)], rows, gs).wait()

        def put(t, sb, ws):
            pltpu.async_copy(sb, out_hbm.at[t, :, pl.ds(b0, 128)], ws)

        def wwait(sb, ws):
            pltpu.make_async_copy(
                sb, out_hbm.at[0, :, pl.ds(b0, 128)], ws
            ).wait()

        def transpose(rows, sb):
            def d_body(k, c):
                for u in range(4):
                    d = k * 4 + u
                    rowv = jnp.bitwise_and(iota + d, DIM - 1)
                    for g in range(8):
                        val = plsc.load_gather(rows, [colgs[g], rowv])
                        plsc.store_scatter(sb, [rowv, colgs[g]], val)
                return c

            lax.fori_loop(0, DIM // 4, d_body, 0)

        rbufs = [rows0, rows1, rows2, rows3]
        gsems = [gs0, gs1, gs2, gs3]
        sbufs = [sb0, sb1]
        wsems = [ws0, ws1]

        for j in range(3):
            fetch(j, rbufs[j], gsems[j])

        def body(k, carry):
            t = k * 4
            for j in range(4):
                nt = t + j + 3

                @pl.when(nt < T)
                def _(nt=nt, j=j):
                    fetch(nt, rbufs[(j + 3) % 4], gsems[(j + 3) % 4])

                if j < 2:

                    @pl.when(k > 0)
                    def _(j=j):
                        wwait(sbufs[j], wsems[j])

                else:
                    wwait(sbufs[j % 2], wsems[j % 2])
                gwait(rbufs[j], gsems[j])
                transpose(rbufs[j], sbufs[j % 2])
                put(t + j, sbufs[j % 2], wsems[j % 2])
            return carry

        lax.fori_loop(0, T // 4, body, 0)
        wwait(sb0, ws0)
        wwait(sb1, ws1)

    return _lookup


def kernel(inp, table):
    ba, t = inp.shape
    idxt = inp.T.astype(jnp.int32)
    tpad = _pad_tc(table.T)
    out3 = _make_lookup(ba, t)(idxt, tpad)
    return jnp.transpose(out3, (2, 0, 1))
